# baseline (speedup 1.0000x reference)
import functools

import jax
import jax.numpy as jnp
from jax.experimental import pallas as pl

LATENT = 64
K = 512
BI = 8


def _vq_body(x_ref, ct_ref, cb_ref, idx_ref, q_ref):
    x = x_ref[...]
    acc = None
    for t in range(8):
        terms = []
        for s in range(8):
            k = 8 * t + s
            xk = x[:, k:k + 1]
            ck = ct_ref[k:k + 1, :]
            d = xk - ck
            terms.append(d * d)
        g = ((terms[0] + terms[4]) + (terms[2] + terms[6])) + (
            (terms[1] + terms[5]) + (terms[3] + terms[7]))
        acc = g if acc is None else acc + g
    dist = acc

    jidx = jax.lax.broadcasted_iota(jnp.int32, (BI, K), 1)
    m = jnp.min(dist, axis=1, keepdims=True)
    idx = jnp.min(jnp.where(dist == m, jidx, K), axis=1)
    idx_ref[0, 0, :] = idx

    onehot = (jidx == idx[:, None]).astype(jnp.float32)
    q_ref[...] = jax.lax.dot_general(
        onehot, cb_ref[...], (((1,), (0,)), ((), ())),
        preferred_element_type=jnp.float32,
        precision=jax.lax.Precision.HIGHEST)


@functools.partial(jax.jit, static_argnames=())
def kernel(inputs, codebook):
    input_shape = inputs.shape
    flat = inputs.reshape(-1, LATENT)
    n = flat.shape[0]
    nblocks = n // BI
    ct = codebook.T

    idx3, quant = pl.pallas_call(
        _vq_body,
        grid=(nblocks,),
        in_specs=[
            pl.BlockSpec((BI, LATENT), lambda i: (i, 0)),
            pl.BlockSpec((LATENT, K), lambda i: (0, 0)),
            pl.BlockSpec((K, LATENT), lambda i: (0, 0)),
        ],
        out_specs=[
            pl.BlockSpec((1, 1, BI), lambda i: (i, 0, 0)),
            pl.BlockSpec((BI, LATENT), lambda i: (i, 0)),
        ],
        out_shape=[
            jax.ShapeDtypeStruct((nblocks, 1, BI), jnp.int32),
            jax.ShapeDtypeStruct((n, LATENT), jnp.float32),
        ],
    )(flat, ct, codebook)

    return idx3.reshape(n), quant.reshape(input_shape)


# speedup vs baseline: 3.6126x; 3.6126x over previous
import functools

import jax
import jax.numpy as jnp
from jax.experimental import pallas as pl
from jax.experimental.pallas import tpu as pltpu

LATENT = 64
K = 512
BI = 128
CH = 8


def _vq_body(x_ref, ct_ref, cb_ref, idx_ref, q_ref, dist_ref):
    for c in range(BI // CH):
        x = x_ref[c * CH:(c + 1) * CH, :]
        acc = None
        for t in range(8):
            terms = []
            for s in range(8):
                k = 8 * t + s
                xk = x[:, k:k + 1]
                ck = ct_ref[k:k + 1, :]
                d = xk - ck
                terms.append(d * d)
            g = ((terms[0] + terms[4]) + (terms[2] + terms[6])) + (
                (terms[1] + terms[5]) + (terms[3] + terms[7]))
            acc = g if acc is None else acc + g
        dist_ref[c * CH:(c + 1) * CH, :] = acc

    dist = dist_ref[...]
    jidx = jax.lax.broadcasted_iota(jnp.int32, (BI, K), 1)
    m = jnp.min(dist, axis=1, keepdims=True)
    idx = jnp.min(jnp.where(dist == m, jidx, K), axis=1)
    idx_ref[0, 0, :] = idx

    onehot = (jidx == idx[:, None]).astype(jnp.float32)
    q_ref[...] = jax.lax.dot_general(
        onehot, cb_ref[...], (((1,), (0,)), ((), ())),
        preferred_element_type=jnp.float32,
        precision=jax.lax.Precision.HIGHEST)


@functools.partial(jax.jit, static_argnames=())
def kernel(inputs, codebook):
    input_shape = inputs.shape
    flat = inputs.reshape(-1, LATENT)
    n = flat.shape[0]
    nblocks = n // BI
    ct = codebook.T

    idx3, quant = pl.pallas_call(
        _vq_body,
        grid=(nblocks,),
        in_specs=[
            pl.BlockSpec((BI, LATENT), lambda i: (i, 0)),
            pl.BlockSpec((LATENT, K), lambda i: (0, 0)),
            pl.BlockSpec((K, LATENT), lambda i: (0, 0)),
        ],
        out_specs=[
            pl.BlockSpec((1, 1, BI), lambda i: (i, 0, 0)),
            pl.BlockSpec((BI, LATENT), lambda i: (i, 0)),
        ],
        out_shape=[
            jax.ShapeDtypeStruct((nblocks, 1, BI), jnp.int32),
            jax.ShapeDtypeStruct((n, LATENT), jnp.float32),
        ],
        scratch_shapes=[pltpu.VMEM((BI, K), jnp.float32)],
    )(flat, ct, codebook)

    return idx3.reshape(n), quant.reshape(input_shape)
